# baseline (speedup 1.0000x reference)
import functools

import jax
import jax.numpy as jnp
from jax import lax
from jax.experimental import pallas as pl
from jax.experimental.pallas import tpu as pltpu
from jax.experimental.pallas import tpu_sc as plsc

_LANES = 16


def _emb_kernel_body(R, C, G, D, num_cores,
                     ids_hbm, vals_hbm, table_hbm, out_hbm,
                     idx_v, blk_v, val_v, rows_v, out_v, sem):
    wid = lax.axis_index("s") * num_cores + lax.axis_index("c")
    base = wid * R
    blocks_per_row = 128 // D

    def chunk_body(g, carry):
        off = base + g * C
        pltpu.sync_copy(ids_hbm.at[pl.ds(off, C)], idx_v)
        def blk_body(ib, c2):
            i0 = ib * _LANES
            idv = idx_v[pl.ds(i0, _LANES)]
            blk_v[pl.ds(i0, _LANES)] = jnp.right_shift(idv, 2)
            return c2
        lax.fori_loop(0, C // _LANES, blk_body, 0)

        copies = []
        for j in range(C // G):
            copies.append(pltpu.async_copy(
                table_hbm.at[blk_v.at[pl.ds(j * G, G)]],
                rows_v.at[pl.ds(j * G, G)], sem))
        pltpu.sync_copy(vals_hbm.at[pl.ds(off, C)], val_v)
        for cp in copies:
            cp.wait()

        def row_body(ib, c2):
            i0 = ib * _LANES
            vv = val_v[pl.ds(i0, _LANES)]
            idv = idx_v[pl.ds(i0, _LANES)]
            qv = jnp.left_shift(jnp.bitwise_and(idv, blocks_per_row - 1),
                                5)
            for k in range(_LANES):
                v = vv[k]
                q = qv[k]
                for h in range(D // _LANES):
                    r = rows_v[i0 + k, pl.ds(q + h * _LANES, _LANES)]
                    out_v[i0 + k, pl.ds(h * _LANES, _LANES)] = r * v
            return c2

        lax.fori_loop(0, C // _LANES, row_body, 0)
        pltpu.sync_copy(out_v, out_hbm.at[pl.ds(off, C)])
        return carry

    lax.fori_loop(0, R // C, chunk_body, 0)


def kernel(feature_id, feature_val, embedding_weight):
    B, F = feature_id.shape
    V, D = embedding_weight.shape
    N = B * F
    ids = feature_id.reshape(N).astype(jnp.int32)
    vals = feature_val.reshape(N)
    table4 = embedding_weight.reshape(V * D // 128, 128)

    info = plsc.get_sparse_core_info()
    NW = info.num_cores * info.num_subcores
    R = N // NW
    C = 256
    G = 128

    mesh = plsc.VectorSubcoreMesh(core_axis_name="c", subcore_axis_name="s")
    body = functools.partial(_emb_kernel_body, R, C, G, D, info.num_cores)
    emb = pl.kernel(
        body,
        mesh=mesh,
        out_type=jax.ShapeDtypeStruct((N, D), jnp.float32),
        scratch_types=[
            pltpu.VMEM((C,), jnp.int32),
            pltpu.VMEM((C,), jnp.int32),
            pltpu.VMEM((C,), jnp.float32),
            pltpu.VMEM((C, 128), jnp.float32),
            pltpu.VMEM((C, D), jnp.float32),
            pltpu.SemaphoreType.DMA,
        ],
    )
    out = emb(ids, vals, table4)
    return out.reshape(B, F, D)


# speedup vs baseline: 1.1680x; 1.1680x over previous
import functools

import jax
import jax.numpy as jnp
from jax import lax
from jax.experimental import pallas as pl
from jax.experimental.pallas import tpu as pltpu
from jax.experimental.pallas import tpu_sc as plsc

_LANES = 16


def _emb_kernel_body(Bp, BC, FG, F, D, num_cores,
                     ids_hbm, vals_hbm, table_hbm, out_hbm,
                     idx_v, val_v, rows_v, out_v, sem):
    wid = lax.axis_index("s") * num_cores + lax.axis_index("c")
    b_base = wid * Bp
    iota = lax.iota(jnp.int32, _LANES)

    def chunk_body(c, carry):
        b0 = b_base + c * BC
        pltpu.sync_copy(ids_hbm.at[:, pl.ds(b0, BC)], idx_v)
        pltpu.sync_copy(vals_hbm.at[:, pl.ds(b0, BC)], val_v)

        def group_body(g, c1):
            copies = []
            for u in range(FG):
                copies.append(pltpu.async_copy(
                    table_hbm.at[idx_v.at[g * FG + u]],
                    rows_v.at[u], sem))
            for cp in copies:
                cp.wait()

            for u in range(FG):
                f = g * FG + u

                def jb_body(jb, c2, u=u, f=f):
                    j0 = jb * _LANES
                    jv = j0 + iota
                    vv = val_v[f, pl.ds(j0, _LANES)]
                    for d in range(D):
                        dv = jnp.full((_LANES,), d, jnp.int32)
                        uv = jnp.full((_LANES,), u, jnp.int32)
                        col = plsc.load_gather(rows_v, [uv, jv, dv])
                        out_v[u, d, pl.ds(j0, _LANES)] = col * vv
                    return c2

                lax.fori_loop(0, BC // _LANES, jb_body, 0)

            pltpu.sync_copy(
                out_v,
                out_hbm.at[pl.ds(g * FG, FG), :, pl.ds(b0, BC)])
            return c1

        lax.fori_loop(0, F // FG, group_body, 0)
        return carry

    lax.fori_loop(0, Bp // BC, chunk_body, 0)


def kernel(feature_id, feature_val, embedding_weight):
    B, F = feature_id.shape
    V, D = embedding_weight.shape
    ids_t = feature_id.T.astype(jnp.int32)
    vals_t = feature_val.T

    info = plsc.get_sparse_core_info()
    NW = info.num_cores * info.num_subcores
    Bp = B // NW
    BC = 128
    FG = 2

    mesh = plsc.VectorSubcoreMesh(core_axis_name="c", subcore_axis_name="s")
    body = functools.partial(_emb_kernel_body, Bp, BC, FG, F, D,
                             info.num_cores)
    emb = pl.kernel(
        body,
        mesh=mesh,
        compiler_params=pltpu.CompilerParams(
            use_tc_tiling_on_sc=False, needs_layout_passes=False),
        out_type=jax.ShapeDtypeStruct((F, D, B), jnp.float32),
        scratch_types=[
            pltpu.VMEM((F, BC), jnp.int32),
            pltpu.VMEM((F, BC), jnp.float32),
            pltpu.VMEM((FG, BC, D), jnp.float32),
            pltpu.VMEM((FG, D, BC), jnp.float32),
            pltpu.SemaphoreType.DMA,
        ],
    )
    out_t = emb(ids_t, vals_t, embedding_weight)
    return jnp.transpose(out_t, (2, 0, 1))


# speedup vs baseline: 1.1746x; 1.0057x over previous
import functools

import jax
import jax.numpy as jnp
from jax import lax
from jax.experimental import pallas as pl
from jax.experimental.pallas import tpu as pltpu
from jax.experimental.pallas import tpu_sc as plsc

_LANES = 16


def _emb_kernel_body(Bp, BC, FG, F, D, num_cores,
                     ids_hbm, vals_hbm, table_hbm, out_hbm,
                     idx_v, val_v, rows_v, out_v, sem):
    wid = lax.axis_index("s") * num_cores + lax.axis_index("c")
    b_base = wid * Bp
    iota = lax.iota(jnp.int32, _LANES)

    def chunk_body(c, carry):
        b0 = b_base + c * BC
        pltpu.sync_copy(ids_hbm.at[:, pl.ds(b0, BC)], idx_v)
        pltpu.sync_copy(vals_hbm.at[:, pl.ds(b0, BC)], val_v)

        def group_body(g, c1):
            copies = []
            for u in range(FG):
                copies.append(pltpu.async_copy(
                    table_hbm.at[idx_v.at[g * FG + u]],
                    rows_v.at[pl.ds(u * BC, BC)], sem))
            for cp in copies:
                cp.wait()

            for u in range(FG):
                f = g * FG + u

                def jb_body(jb, c2, u=u, f=f):
                    j0 = jb * _LANES
                    rowv = u * BC + j0 + iota
                    vv = val_v[f, pl.ds(j0, _LANES)]
                    for d in range(D):
                        dv = jnp.full((_LANES,), d, jnp.int32)
                        col = plsc.load_gather(rows_v, [rowv, dv])
                        out_v[u, d, pl.ds(j0, _LANES)] = col * vv
                    return c2

                lax.fori_loop(0, BC // _LANES, jb_body, 0)

            pltpu.sync_copy(
                out_v,
                out_hbm.at[pl.ds(g * FG, FG), :, pl.ds(b0, BC)])
            return c1

        lax.fori_loop(0, F // FG, group_body, 0)
        return carry

    lax.fori_loop(0, Bp // BC, chunk_body, 0)


def kernel(feature_id, feature_val, embedding_weight):
    B, F = feature_id.shape
    V, D = embedding_weight.shape
    ids_t = feature_id.T.astype(jnp.int32)
    vals_t = feature_val.T

    info = plsc.get_sparse_core_info()
    NW = info.num_cores * info.num_subcores
    Bp = B // NW
    BC = 128
    FG = 2

    mesh = plsc.VectorSubcoreMesh(core_axis_name="c", subcore_axis_name="s")
    body = functools.partial(_emb_kernel_body, Bp, BC, FG, F, D,
                             info.num_cores)
    emb = pl.kernel(
        body,
        mesh=mesh,
        compiler_params=pltpu.CompilerParams(
            use_tc_tiling_on_sc=False, needs_layout_passes=False),
        out_type=jax.ShapeDtypeStruct((F, D, B), jnp.float32),
        scratch_types=[
            pltpu.VMEM((F, BC), jnp.int32),
            pltpu.VMEM((F, BC), jnp.float32),
            pltpu.VMEM((FG * BC, D), jnp.float32),
            pltpu.VMEM((FG, D, BC), jnp.float32),
            pltpu.SemaphoreType.DMA,
        ],
    )
    out_t = emb(ids_t, vals_t, embedding_weight)
    return jnp.transpose(out_t, (2, 0, 1))


# speedup vs baseline: 1.4481x; 1.2328x over previous
import functools

import jax
import jax.numpy as jnp
from jax import lax
from jax.experimental import pallas as pl
from jax.experimental.pallas import tpu as pltpu
from jax.experimental.pallas import tpu_sc as plsc

_LANES = 16


def _emb_kernel_body(Bp, BC, F, D, num_cores,
                     ids_hbm, vals_hbm, table_hbm, out_hbm,
                     idx_v, val_v, rows_v, gsem, osem):
    wid = lax.axis_index("s") * num_cores + lax.axis_index("c")
    b_base = wid * Bp

    def gather_desc(f, u):
        return pltpu.make_async_copy(
            table_hbm.at[idx_v.at[f]],
            rows_v.at[pl.ds(u * BC, BC)], gsem)

    def out_desc(f, u, b0):
        return pltpu.make_async_copy(
            rows_v.at[pl.ds(u * BC, BC)],
            out_hbm.at[f, pl.ds(b0, BC), :], osem)

    def chunk_body(c, carry):
        b0 = b_base + c * BC
        pltpu.sync_copy(ids_hbm.at[:, pl.ds(b0, BC)], idx_v)
        pltpu.sync_copy(vals_hbm.at[:, pl.ds(b0, BC)], val_v)

        gather_desc(0, 0).start()

        def pair_body(g, c1):
            for u in range(2):
                f = 2 * g + u
                gather_desc(f, u).wait()
                if u == 1:
                    out_desc(f - 1, 0, b0).wait()
                else:
                    @pl.when(g > 0)
                    def _():
                        out_desc(f - 1, 1, b0).wait()

                @pl.when(f + 1 < F)
                def _():
                    gather_desc(f + 1, 1 - u).start()

                def jb_body(jb, c2, u=u, f=f):
                    j0 = jb * _LANES
                    vv = val_v[f, pl.ds(j0, _LANES)]
                    for k in range(_LANES):
                        r = u * BC + j0 + k
                        v = vv[k]
                        for h in range(D // _LANES):
                            x = rows_v[r, pl.ds(h * _LANES, _LANES)]
                            rows_v[r, pl.ds(h * _LANES, _LANES)] = x * v
                    return c2

                lax.fori_loop(0, BC // _LANES, jb_body, 0)
                out_desc(f, u, b0).start()
            return c1

        lax.fori_loop(0, F // 2, pair_body, 0)
        out_desc(F - 1, 1, b0).wait()
        return carry

    lax.fori_loop(0, Bp // BC, chunk_body, 0)


def kernel(feature_id, feature_val, embedding_weight):
    B, F = feature_id.shape
    V, D = embedding_weight.shape
    ids_t = feature_id.T.astype(jnp.int32)
    vals_t = feature_val.T

    info = plsc.get_sparse_core_info()
    NW = info.num_cores * info.num_subcores
    Bp = B // NW
    BC = 128

    mesh = plsc.VectorSubcoreMesh(core_axis_name="c", subcore_axis_name="s")
    body = functools.partial(_emb_kernel_body, Bp, BC, F, D, info.num_cores)
    emb = pl.kernel(
        body,
        mesh=mesh,
        compiler_params=pltpu.CompilerParams(use_tc_tiling_on_sc=False),
        out_type=jax.ShapeDtypeStruct((F, B, D), jnp.float32),
        scratch_types=[
            pltpu.VMEM((F, BC), jnp.int32),
            pltpu.VMEM((F, BC), jnp.float32),
            pltpu.VMEM((2 * BC, D), jnp.float32),
            pltpu.SemaphoreType.DMA,
            pltpu.SemaphoreType.DMA,
        ],
    )
    out_t = emb(ids_t, vals_t, embedding_weight)
    return jnp.transpose(out_t, (1, 0, 2))
